# Initial kernel scaffold; baseline (speedup 1.0000x reference)
import functools

import jax
import jax.numpy as jnp
from jax import lax
from jax.experimental import pallas as pl
from jax.experimental.pallas import tpu as pltpu
from jax.experimental.pallas import tpu_sc as plsc

N = 50000
K = 16
E = N * K
TN = 400
RT = TN * K
G = N // TN

_NC = 2
_NS = 16
_NW = _NC * _NS
_CH = 1000

_HI = jax.lax.Precision.HIGHEST


def _sc_gather(table, idx):
    e = idx.shape[0]
    d = table.shape[1]
    bpw = e // _NW
    nch = bpw // _CH
    mesh = plsc.VectorSubcoreMesh(core_axis_name="c", subcore_axis_name="s")

    @functools.partial(
        pl.kernel,
        mesh=mesh,
        out_type=jax.ShapeDtypeStruct((e, d), jnp.float32),
        scratch_types=[
            pltpu.VMEM((_CH,), jnp.int32),
            pltpu.VMEM((_CH, d), jnp.float32),
            pltpu.SemaphoreType.DMA,
        ],
    )
    def k(table_hbm, idx_hbm, out_hbm, idx_v, rows_v, sem):
        wid = lax.axis_index("s") * _NC + lax.axis_index("c")
        base = wid * bpw

        def body(c, carry):
            off = base + c * _CH
            pltpu.sync_copy(idx_hbm.at[pl.ds(off, _CH)], idx_v)
            pltpu.async_copy(table_hbm.at[idx_v], rows_v, sem).wait()
            pltpu.sync_copy(rows_v, out_hbm.at[pl.ds(off, _CH)])
            return carry

        lax.fori_loop(0, nch, body, 0)

    return k(table, idx)


def _rel_pos_block(g, xyz_blk):
    xj = g[:, 0:3]
    xi = jnp.broadcast_to(xyz_blk[:, None, :], (TN, K, 3)).reshape(RT, 3)
    rel = xi - xj
    dis = jnp.sqrt(jnp.sum(rel * rel, axis=1, keepdims=True))
    return jnp.concatenate([dis, rel, xi, xj], axis=1)


def _stats_accum(ref, vals):
    s = jnp.sum(vals, axis=0, keepdims=True)
    q = jnp.sum(vals * vals, axis=0, keepdims=True)
    part = jnp.broadcast_to(jnp.concatenate([s, q], axis=1), ref.shape)
    i = pl.program_id(0)

    @pl.when(i == 0)
    def _():
        ref[...] = part

    @pl.when(i > 0)
    def _():
        ref[...] = ref[...] + part


def _tc1_body(gath_ref, xyz_ref, w1_ref, b1_ref, stats_ref):
    rp = _rel_pos_block(gath_ref[...], xyz_ref[...])
    y1 = jnp.dot(rp, w1_ref[...].T, precision=_HI) + b1_ref[0:1, :]
    _stats_accum(stats_ref, y1)


def _softmax_pool(f_cat, att):
    a3 = att.reshape(TN, K, att.shape[1])
    f3 = f_cat.reshape(TN, K, f_cat.shape[1])
    m = jnp.max(a3, axis=1, keepdims=True)
    ex = jnp.exp(a3 - m)
    sm = jnp.sum(ex, axis=1, keepdims=True)
    return jnp.sum(f3 * (ex / sm), axis=1)


def _tc2_body(gath_ref, xyz_ref, w1_ref, b1_ref, aff1_ref, aw1_ref, ab1_ref,
              mw1_ref, mb1_ref, w2_ref, b2_ref,
              ym1_ref, y2_ref, stats_m1_ref, stats_y2_ref):
    g = gath_ref[...]
    rp = _rel_pos_block(g, xyz_ref[...])
    y1 = jnp.dot(rp, w1_ref[...].T, precision=_HI) + b1_ref[0:1, :]
    f_xyz = jnp.maximum(y1 * aff1_ref[0:1, :] + aff1_ref[1:2, :], 0.0)
    f_cat = jnp.concatenate([g[:, 16:32], f_xyz], axis=1)
    att = jnp.dot(f_cat, aw1_ref[...].T, precision=_HI) + ab1_ref[0:1, :]
    agg = _softmax_pool(f_cat, att)
    y_m1 = jnp.dot(agg, mw1_ref[...].T, precision=_HI) + mb1_ref[0:1, :]
    ym1_ref[...] = y_m1
    y2 = jnp.dot(f_xyz, w2_ref[...].T, precision=_HI) + b2_ref[0:1, :]
    y2_ref[...] = y2
    _stats_accum(stats_m1_ref, y_m1)
    _stats_accum(stats_y2_ref, y2)


def _tc3_body(y2_ref, gath2_ref, aff2_ref, affm1_ref, aw2_ref, ab2_ref,
              mw2_ref, mb2_ref, ym2_ref, stats_m2_ref):
    f_xyz2 = jnp.maximum(y2_ref[...] * aff2_ref[0:1, :] + aff2_ref[1:2, :], 0.0)
    f_nb2 = jnp.maximum(gath2_ref[...] * affm1_ref[0:1, :] + affm1_ref[1:2, :], 0.0)
    f_cat2 = jnp.concatenate([f_nb2, f_xyz2], axis=1)
    att2 = jnp.dot(f_cat2, aw2_ref[...].T, precision=_HI) + ab2_ref[0:1, :]
    agg2 = _softmax_pool(f_cat2, att2)
    y_m2 = jnp.dot(agg2, mw2_ref[...].T, precision=_HI) + mb2_ref[0:1, :]
    ym2_ref[...] = y_m2
    _stats_accum(stats_m2_ref, y_m2)


def _tc4_body(ym2_ref, affm2_ref, out_ref):
    out_ref[...] = jnp.maximum(
        ym2_ref[...] * affm2_ref[0:1, :] + affm2_ref[1:2, :], 0.0)


def _full(shape):
    return pl.BlockSpec(shape, lambda i: (0, 0))


def kernel(xyz, feature, neigh_idx, W1, b1, g1, be1, aW1, ab1, mW1, mb1,
           mg1, mbe1, W2, b2, g2, be2, aW2, ab2, mW2, mb2, mg2, mbe2):
    f32 = jnp.float32
    P = xyz[0].astype(f32)
    feat = feature[0, :, :, 0].T.astype(f32)
    table1 = jnp.concatenate(
        [P, jnp.zeros((N, 13), f32), feat], axis=1)
    idx = neigh_idx.reshape(E).astype(jnp.int32)

    gath1 = _sc_gather(table1, idx)

    b1r = jnp.broadcast_to(b1, (8, 16))
    stats1 = pl.pallas_call(
        _tc1_body,
        grid=(G,),
        in_specs=[
            pl.BlockSpec((RT, 32), lambda i: (i, 0)),
            pl.BlockSpec((TN, 3), lambda i: (i, 0)),
            _full((16, 10)),
            _full((8, 16)),
        ],
        out_specs=_full((8, 32)),
        out_shape=jax.ShapeDtypeStruct((8, 32), f32),
    )(gath1, P, W1, b1r)

    s1, q1 = stats1[0, :16], stats1[0, 16:32]
    m1 = s1 / E
    v1 = q1 / E - m1 * m1
    sc1 = g1 / jnp.sqrt(v1 + 1e-5)
    aff1 = jnp.stack([sc1, be1 - m1 * sc1])

    y_m1, y2, stats_m1, stats_y2 = pl.pallas_call(
        _tc2_body,
        grid=(G,),
        in_specs=[
            pl.BlockSpec((RT, 32), lambda i: (i, 0)),
            pl.BlockSpec((TN, 3), lambda i: (i, 0)),
            _full((16, 10)),
            _full((8, 16)),
            _full((2, 16)),
            _full((32, 32)),
            _full((8, 32)),
            _full((16, 32)),
            _full((8, 16)),
            _full((16, 16)),
            _full((8, 16)),
        ],
        out_specs=[
            pl.BlockSpec((TN, 16), lambda i: (i, 0)),
            pl.BlockSpec((RT, 16), lambda i: (i, 0)),
            _full((8, 32)),
            _full((8, 32)),
        ],
        out_shape=[
            jax.ShapeDtypeStruct((N, 16), f32),
            jax.ShapeDtypeStruct((E, 16), f32),
            jax.ShapeDtypeStruct((8, 32), f32),
            jax.ShapeDtypeStruct((8, 32), f32),
        ],
    )(gath1, P, W1, b1r, aff1, aW1, jnp.broadcast_to(ab1, (8, 32)),
      mW1, jnp.broadcast_to(mb1, (8, 16)), W2, jnp.broadcast_to(b2, (8, 16)))

    sm, qm = stats_m1[0, :16], stats_m1[0, 16:32]
    mm = sm / N
    vm = qm / N - mm * mm
    scm = mg1 / jnp.sqrt(vm + 1e-5)
    affm1 = jnp.stack([scm, mbe1 - mm * scm])

    s2, q2 = stats_y2[0, :16], stats_y2[0, 16:32]
    m2 = s2 / E
    v2 = q2 / E - m2 * m2
    sc2 = g2 / jnp.sqrt(v2 + 1e-5)
    aff2 = jnp.stack([sc2, be2 - m2 * sc2])

    gath2 = _sc_gather(y_m1, idx)

    y_m2, stats_m2 = pl.pallas_call(
        _tc3_body,
        grid=(G,),
        in_specs=[
            pl.BlockSpec((RT, 16), lambda i: (i, 0)),
            pl.BlockSpec((RT, 16), lambda i: (i, 0)),
            _full((2, 16)),
            _full((2, 16)),
            _full((32, 32)),
            _full((8, 32)),
            _full((32, 32)),
            _full((8, 32)),
        ],
        out_specs=[
            pl.BlockSpec((TN, 32), lambda i: (i, 0)),
            _full((8, 64)),
        ],
        out_shape=[
            jax.ShapeDtypeStruct((N, 32), f32),
            jax.ShapeDtypeStruct((8, 64), f32),
        ],
    )(y2, gath2, aff2, affm1, aW2, jnp.broadcast_to(ab2, (8, 32)),
      mW2, jnp.broadcast_to(mb2, (8, 32)))

    sm2, qm2 = stats_m2[0, :32], stats_m2[0, 32:64]
    mm2 = sm2 / N
    vm2 = qm2 / N - mm2 * mm2
    scm2 = mg2 / jnp.sqrt(vm2 + 1e-5)
    affm2 = jnp.stack([scm2, mbe2 - mm2 * scm2])

    out = pl.pallas_call(
        _tc4_body,
        grid=(G,),
        in_specs=[
            pl.BlockSpec((TN, 32), lambda i: (i, 0)),
            _full((2, 32)),
        ],
        out_specs=pl.BlockSpec((TN, 32), lambda i: (i, 0)),
        out_shape=jax.ShapeDtypeStruct((N, 32), f32),
    )(y_m2, affm2)

    return out.T.reshape(1, 32, N, 1)


# speedup vs baseline: 3.7516x; 3.7516x over previous
import functools

import jax
import jax.numpy as jnp
from jax import lax
from jax.experimental import pallas as pl
from jax.experimental.pallas import tpu as pltpu
from jax.experimental.pallas import tpu_sc as plsc

N = 50000
K = 16
E = N * K
TN = 400
RT = TN * K
G = N // TN

_NC = 2
_NS = 16
_NW = _NC * _NS
_CH = 1000

_HI = jax.lax.Precision.HIGHEST


def _sc_gather(table, idx):
    e = idx.shape[0]
    d = table.shape[1]
    bpw = e // _NW
    nch = bpw // _CH
    mesh = plsc.VectorSubcoreMesh(core_axis_name="c", subcore_axis_name="s")

    @functools.partial(
        pl.kernel,
        mesh=mesh,
        out_type=jax.ShapeDtypeStruct((e, d), jnp.float32),
        compiler_params=pltpu.CompilerParams(use_tc_tiling_on_sc=False),
        scratch_types=[
            pltpu.VMEM((_CH,), jnp.int32),
            pltpu.VMEM((_CH, d), jnp.float32),
            pltpu.SemaphoreType.DMA,
        ],
    )
    def k(table_hbm, idx_hbm, out_hbm, idx_v, rows_v, sem):
        wid = lax.axis_index("s") * _NC + lax.axis_index("c")
        base = wid * bpw

        def body(c, carry):
            off = base + c * _CH
            pltpu.sync_copy(idx_hbm.at[pl.ds(off, _CH)], idx_v)
            pltpu.async_copy(table_hbm.at[idx_v], rows_v, sem).wait()
            pltpu.sync_copy(rows_v, out_hbm.at[pl.ds(off, _CH)])
            return carry

        lax.fori_loop(0, nch, body, 0)

    return k(table, idx)


def _rel_pos_block(g, xyz_blk):
    xj = g[:, 0:3]
    xi = jnp.broadcast_to(xyz_blk[:, None, :], (TN, K, 3)).reshape(RT, 3)
    rel = xi - xj
    dis = jnp.sqrt(jnp.sum(rel * rel, axis=1, keepdims=True))
    return jnp.concatenate([dis, rel, xi, xj], axis=1)


def _stats_accum(ref, vals):
    s = jnp.sum(vals, axis=0, keepdims=True)
    q = jnp.sum(vals * vals, axis=0, keepdims=True)
    part = jnp.broadcast_to(jnp.concatenate([s, q], axis=1), ref.shape)
    i = pl.program_id(0)

    @pl.when(i == 0)
    def _():
        ref[...] = part

    @pl.when(i > 0)
    def _():
        ref[...] = ref[...] + part


def _tc1_body(gath_ref, xyz_ref, w1_ref, b1_ref, stats_ref):
    rp = _rel_pos_block(gath_ref[...], xyz_ref[...])
    y1 = jnp.dot(rp, w1_ref[...].T, precision=_HI) + b1_ref[0:1, :]
    _stats_accum(stats_ref, y1)


def _softmax_pool(f_cat, att):
    a3 = att.reshape(TN, K, att.shape[1])
    f3 = f_cat.reshape(TN, K, f_cat.shape[1])
    m = jnp.max(a3, axis=1, keepdims=True)
    ex = jnp.exp(a3 - m)
    sm = jnp.sum(ex, axis=1, keepdims=True)
    return jnp.sum(f3 * (ex / sm), axis=1)


def _tc2_body(gath_ref, xyz_ref, w1_ref, b1_ref, aff1_ref, aw1_ref, ab1_ref,
              mw1_ref, mb1_ref, w2_ref, b2_ref,
              ym1_ref, y2_ref, stats_m1_ref, stats_y2_ref):
    g = gath_ref[...]
    rp = _rel_pos_block(g, xyz_ref[...])
    y1 = jnp.dot(rp, w1_ref[...].T, precision=_HI) + b1_ref[0:1, :]
    f_xyz = jnp.maximum(y1 * aff1_ref[0:1, :] + aff1_ref[1:2, :], 0.0)
    f_cat = jnp.concatenate([g[:, 16:32], f_xyz], axis=1)
    att = jnp.dot(f_cat, aw1_ref[...].T, precision=_HI) + ab1_ref[0:1, :]
    agg = _softmax_pool(f_cat, att)
    y_m1 = jnp.dot(agg, mw1_ref[...].T, precision=_HI) + mb1_ref[0:1, :]
    ym1_ref[...] = y_m1
    y2 = jnp.dot(f_xyz, w2_ref[...].T, precision=_HI) + b2_ref[0:1, :]
    y2_ref[...] = y2
    _stats_accum(stats_m1_ref, y_m1)
    _stats_accum(stats_y2_ref, y2)


def _tc3_body(y2_ref, gath2_ref, aff2_ref, affm1_ref, aw2_ref, ab2_ref,
              mw2_ref, mb2_ref, ym2_ref, stats_m2_ref):
    f_xyz2 = jnp.maximum(y2_ref[...] * aff2_ref[0:1, :] + aff2_ref[1:2, :], 0.0)
    f_nb2 = jnp.maximum(gath2_ref[...] * affm1_ref[0:1, :] + affm1_ref[1:2, :], 0.0)
    f_cat2 = jnp.concatenate([f_nb2, f_xyz2], axis=1)
    att2 = jnp.dot(f_cat2, aw2_ref[...].T, precision=_HI) + ab2_ref[0:1, :]
    agg2 = _softmax_pool(f_cat2, att2)
    y_m2 = jnp.dot(agg2, mw2_ref[...].T, precision=_HI) + mb2_ref[0:1, :]
    ym2_ref[...] = y_m2
    _stats_accum(stats_m2_ref, y_m2)


def _tc4_body(ym2_ref, affm2_ref, out_ref):
    out_ref[...] = jnp.maximum(
        ym2_ref[...] * affm2_ref[0:1, :] + affm2_ref[1:2, :], 0.0)


def _full(shape):
    return pl.BlockSpec(shape, lambda i: (0, 0))


def kernel(xyz, feature, neigh_idx, W1, b1, g1, be1, aW1, ab1, mW1, mb1,
           mg1, mbe1, W2, b2, g2, be2, aW2, ab2, mW2, mb2, mg2, mbe2):
    f32 = jnp.float32
    P = xyz[0].astype(f32)
    feat = feature[0, :, :, 0].T.astype(f32)
    table1 = jnp.concatenate(
        [P, jnp.zeros((N, 13), f32), feat], axis=1)
    idx = neigh_idx.reshape(E).astype(jnp.int32)

    gath1 = _sc_gather(table1, idx)

    b1r = jnp.broadcast_to(b1, (8, 16))
    stats1 = pl.pallas_call(
        _tc1_body,
        grid=(G,),
        in_specs=[
            pl.BlockSpec((RT, 32), lambda i: (i, 0)),
            pl.BlockSpec((TN, 3), lambda i: (i, 0)),
            _full((16, 10)),
            _full((8, 16)),
        ],
        out_specs=_full((8, 32)),
        out_shape=jax.ShapeDtypeStruct((8, 32), f32),
    )(gath1, P, W1, b1r)

    s1, q1 = stats1[0, :16], stats1[0, 16:32]
    m1 = s1 / E
    v1 = q1 / E - m1 * m1
    sc1 = g1 / jnp.sqrt(v1 + 1e-5)
    aff1 = jnp.stack([sc1, be1 - m1 * sc1])

    y_m1, y2, stats_m1, stats_y2 = pl.pallas_call(
        _tc2_body,
        grid=(G,),
        in_specs=[
            pl.BlockSpec((RT, 32), lambda i: (i, 0)),
            pl.BlockSpec((TN, 3), lambda i: (i, 0)),
            _full((16, 10)),
            _full((8, 16)),
            _full((2, 16)),
            _full((32, 32)),
            _full((8, 32)),
            _full((16, 32)),
            _full((8, 16)),
            _full((16, 16)),
            _full((8, 16)),
        ],
        out_specs=[
            pl.BlockSpec((TN, 16), lambda i: (i, 0)),
            pl.BlockSpec((RT, 16), lambda i: (i, 0)),
            _full((8, 32)),
            _full((8, 32)),
        ],
        out_shape=[
            jax.ShapeDtypeStruct((N, 16), f32),
            jax.ShapeDtypeStruct((E, 16), f32),
            jax.ShapeDtypeStruct((8, 32), f32),
            jax.ShapeDtypeStruct((8, 32), f32),
        ],
    )(gath1, P, W1, b1r, aff1, aW1, jnp.broadcast_to(ab1, (8, 32)),
      mW1, jnp.broadcast_to(mb1, (8, 16)), W2, jnp.broadcast_to(b2, (8, 16)))

    sm, qm = stats_m1[0, :16], stats_m1[0, 16:32]
    mm = sm / N
    vm = qm / N - mm * mm
    scm = mg1 / jnp.sqrt(vm + 1e-5)
    affm1 = jnp.stack([scm, mbe1 - mm * scm])

    s2, q2 = stats_y2[0, :16], stats_y2[0, 16:32]
    m2 = s2 / E
    v2 = q2 / E - m2 * m2
    sc2 = g2 / jnp.sqrt(v2 + 1e-5)
    aff2 = jnp.stack([sc2, be2 - m2 * sc2])

    gath2 = _sc_gather(y_m1, idx)

    y_m2, stats_m2 = pl.pallas_call(
        _tc3_body,
        grid=(G,),
        in_specs=[
            pl.BlockSpec((RT, 16), lambda i: (i, 0)),
            pl.BlockSpec((RT, 16), lambda i: (i, 0)),
            _full((2, 16)),
            _full((2, 16)),
            _full((32, 32)),
            _full((8, 32)),
            _full((32, 32)),
            _full((8, 32)),
        ],
        out_specs=[
            pl.BlockSpec((TN, 32), lambda i: (i, 0)),
            _full((8, 64)),
        ],
        out_shape=[
            jax.ShapeDtypeStruct((N, 32), f32),
            jax.ShapeDtypeStruct((8, 64), f32),
        ],
    )(y2, gath2, aff2, affm1, aW2, jnp.broadcast_to(ab2, (8, 32)),
      mW2, jnp.broadcast_to(mb2, (8, 32)))

    sm2, qm2 = stats_m2[0, :32], stats_m2[0, 32:64]
    mm2 = sm2 / N
    vm2 = qm2 / N - mm2 * mm2
    scm2 = mg2 / jnp.sqrt(vm2 + 1e-5)
    affm2 = jnp.stack([scm2, mbe2 - mm2 * scm2])

    out = pl.pallas_call(
        _tc4_body,
        grid=(G,),
        in_specs=[
            pl.BlockSpec((TN, 32), lambda i: (i, 0)),
            _full((2, 32)),
        ],
        out_specs=pl.BlockSpec((TN, 32), lambda i: (i, 0)),
        out_shape=jax.ShapeDtypeStruct((N, 32), f32),
    )(y_m2, affm2)

    return out.T.reshape(1, 32, N, 1)


# speedup vs baseline: 10.2334x; 2.7278x over previous
import functools

import jax
import jax.numpy as jnp
from jax import lax
from jax.experimental import pallas as pl
from jax.experimental.pallas import tpu as pltpu
from jax.experimental.pallas import tpu_sc as plsc

N = 50000
K = 16
E = N * K
TN = 400
RT = TN * K
G = N // TN

_NC = 2
_NS = 16
_NW = _NC * _NS
_CH = 1000

_HI = jax.lax.Precision.HIGHEST
_LO = jax.lax.Precision.DEFAULT


def _sc_gather(table, idx):
    e = idx.shape[0]
    d = table.shape[1]
    bpw = e // _NW
    nch = bpw // _CH
    mesh = plsc.VectorSubcoreMesh(core_axis_name="c", subcore_axis_name="s")

    @functools.partial(
        pl.kernel,
        mesh=mesh,
        out_type=jax.ShapeDtypeStruct((e, d), jnp.float32),
        compiler_params=pltpu.CompilerParams(use_tc_tiling_on_sc=False),
        scratch_types=[
            pltpu.VMEM((_CH,), jnp.int32),
            pltpu.VMEM((_CH, d), jnp.float32),
            pltpu.SemaphoreType.DMA,
        ],
    )
    def k(table_hbm, idx_hbm, out_hbm, idx_v, rows_v, sem):
        wid = lax.axis_index("s") * _NC + lax.axis_index("c")
        base = wid * bpw

        def body(c, carry):
            off = base + c * _CH
            pltpu.sync_copy(idx_hbm.at[pl.ds(off, _CH)], idx_v)
            pltpu.async_copy(table_hbm.at[idx_v], rows_v, sem).wait()
            pltpu.sync_copy(rows_v, out_hbm.at[pl.ds(off, _CH)])
            return carry

        lax.fori_loop(0, nch, body, 0)

    return k(table, idx)


def _stats_accum(ref, vals):
    s = jnp.sum(vals, axis=0, keepdims=True)
    q = jnp.sum(vals * vals, axis=0, keepdims=True)
    part = jnp.broadcast_to(jnp.concatenate([s, q], axis=1), ref.shape)
    i = pl.program_id(0)

    @pl.when(i == 0)
    def _():
        ref[...] = part

    @pl.when(i > 0)
    def _():
        ref[...] = ref[...] + part


def _tc1_body(gath_ref, xyz_ref, w1t_ref, b1_ref, y1_ref, stats_ref):
    g = gath_ref[...]
    xj = g[:, 0:3]
    xi = jnp.broadcast_to(xyz_ref[...][:, None, :], (TN, K, 3)).reshape(RT, 3)
    rel = xi - xj
    dis = jnp.sqrt(jnp.sum(rel * rel, axis=1, keepdims=True))
    rp = jnp.concatenate([dis, rel, xi, xj], axis=1)
    y1 = jnp.dot(rp, w1t_ref[...], precision=_LO) + b1_ref[0:1, :]
    y1_ref[...] = y1
    _stats_accum(stats_ref, y1)


def _softmax_pool(f_cat, att):
    a3 = att.reshape(TN, K, att.shape[1])
    f3 = f_cat.reshape(TN, K, f_cat.shape[1])
    m = jnp.max(a3, axis=1, keepdims=True)
    ex = jnp.exp(a3 - m)
    sm = jnp.sum(ex, axis=1, keepdims=True)
    return jnp.sum(f3 * (ex / sm), axis=1)


def _tc2_body(y1_ref, gath_ref, aff1_ref, awc_ref, abc_ref, mw1t_ref, mb1_ref,
              ym1_ref, y2_ref, stats_m1_ref, stats_y2_ref):
    f_xyz = jnp.maximum(
        y1_ref[...] * aff1_ref[0:1, :] + aff1_ref[1:2, :], 0.0)
    f_cat = jnp.concatenate([gath_ref[...][:, 16:32], f_xyz], axis=1)
    av = jnp.dot(f_cat, awc_ref[...], precision=_LO) + abc_ref[0:1, :]
    att = av[:, 0:32]
    y2 = av[:, 32:48]
    agg = _softmax_pool(f_cat, att)
    y_m1 = jnp.dot(agg, mw1t_ref[...], precision=_HI) + mb1_ref[0:1, :]
    ym1_ref[...] = y_m1
    y2_ref[...] = y2
    _stats_accum(stats_m1_ref, y_m1)
    _stats_accum(stats_y2_ref, y2)


def _tc3_body(y2_ref, gath2_ref, aff2_ref, affm1_ref, aw2t_ref, ab2_ref,
              mw2t_ref, mb2_ref, ym2_ref, stats_m2_ref):
    f_xyz2 = jnp.maximum(y2_ref[...] * aff2_ref[0:1, :] + aff2_ref[1:2, :], 0.0)
    f_nb2 = jnp.maximum(gath2_ref[...] * affm1_ref[0:1, :] + affm1_ref[1:2, :], 0.0)
    f_cat2 = jnp.concatenate([f_nb2, f_xyz2], axis=1)
    att2 = jnp.dot(f_cat2, aw2t_ref[...], precision=_LO) + ab2_ref[0:1, :]
    agg2 = _softmax_pool(f_cat2, att2)
    y_m2 = jnp.dot(agg2, mw2t_ref[...], precision=_HI) + mb2_ref[0:1, :]
    ym2_ref[...] = y_m2
    _stats_accum(stats_m2_ref, y_m2)


def _tc4_body(ym2_ref, affm2_ref, out_ref):
    out_ref[...] = jnp.maximum(
        ym2_ref[...] * affm2_ref[0:1, :] + affm2_ref[1:2, :], 0.0)


def _full(shape):
    return pl.BlockSpec(shape, lambda i: (0, 0))


def kernel(xyz, feature, neigh_idx, W1, b1, g1, be1, aW1, ab1, mW1, mb1,
           mg1, mbe1, W2, b2, g2, be2, aW2, ab2, mW2, mb2, mg2, mbe2):
    f32 = jnp.float32
    P = xyz[0].astype(f32)
    feat = feature[0, :, :, 0].T.astype(f32)
    table1 = jnp.concatenate(
        [P, jnp.zeros((N, 13), f32), feat], axis=1)
    idx = neigh_idx.reshape(E).astype(jnp.int32)

    gath1 = _sc_gather(table1, idx)

    y1, stats1 = pl.pallas_call(
        _tc1_body,
        grid=(G,),
        in_specs=[
            pl.BlockSpec((RT, 32), lambda i: (i, 0)),
            pl.BlockSpec((TN, 3), lambda i: (i, 0)),
            _full((10, 16)),
            _full((8, 16)),
        ],
        out_specs=[
            pl.BlockSpec((RT, 16), lambda i: (i, 0)),
            _full((8, 32)),
        ],
        out_shape=[
            jax.ShapeDtypeStruct((E, 16), f32),
            jax.ShapeDtypeStruct((8, 32), f32),
        ],
    )(gath1, P, W1.T, jnp.broadcast_to(b1, (8, 16)))

    s1, q1 = stats1[0, :16], stats1[0, 16:32]
    m1 = s1 / E
    v1 = q1 / E - m1 * m1
    sc1 = g1 / jnp.sqrt(v1 + 1e-5)
    aff1 = jnp.stack([sc1, be1 - m1 * sc1])

    aWc = jnp.concatenate(
        [aW1.T, jnp.concatenate([jnp.zeros((16, 16), f32), W2.T], axis=0)],
        axis=1)
    abc = jnp.concatenate([ab1, b2])

    y_m1, y2, stats_m1, stats_y2 = pl.pallas_call(
        _tc2_body,
        grid=(G,),
        in_specs=[
            pl.BlockSpec((RT, 16), lambda i: (i, 0)),
            pl.BlockSpec((RT, 32), lambda i: (i, 0)),
            _full((2, 16)),
            _full((32, 48)),
            _full((8, 48)),
            _full((32, 16)),
            _full((8, 16)),
        ],
        out_specs=[
            pl.BlockSpec((TN, 16), lambda i: (i, 0)),
            pl.BlockSpec((RT, 16), lambda i: (i, 0)),
            _full((8, 32)),
            _full((8, 32)),
        ],
        out_shape=[
            jax.ShapeDtypeStruct((N, 16), f32),
            jax.ShapeDtypeStruct((E, 16), f32),
            jax.ShapeDtypeStruct((8, 32), f32),
            jax.ShapeDtypeStruct((8, 32), f32),
        ],
    )(y1, gath1, aff1, aWc, jnp.broadcast_to(abc, (8, 48)),
      mW1.T, jnp.broadcast_to(mb1, (8, 16)))

    sm, qm = stats_m1[0, :16], stats_m1[0, 16:32]
    mm = sm / N
    vm = qm / N - mm * mm
    scm = mg1 / jnp.sqrt(vm + 1e-5)
    affm1 = jnp.stack([scm, mbe1 - mm * scm])

    s2, q2 = stats_y2[0, :16], stats_y2[0, 16:32]
    m2 = s2 / E
    v2 = q2 / E - m2 * m2
    sc2 = g2 / jnp.sqrt(v2 + 1e-5)
    aff2 = jnp.stack([sc2, be2 - m2 * sc2])

    gath2 = _sc_gather(y_m1, idx)

    y_m2, stats_m2 = pl.pallas_call(
        _tc3_body,
        grid=(G,),
        in_specs=[
            pl.BlockSpec((RT, 16), lambda i: (i, 0)),
            pl.BlockSpec((RT, 16), lambda i: (i, 0)),
            _full((2, 16)),
            _full((2, 16)),
            _full((32, 32)),
            _full((8, 32)),
            _full((32, 32)),
            _full((8, 32)),
        ],
        out_specs=[
            pl.BlockSpec((TN, 32), lambda i: (i, 0)),
            _full((8, 64)),
        ],
        out_shape=[
            jax.ShapeDtypeStruct((N, 32), f32),
            jax.ShapeDtypeStruct((8, 64), f32),
        ],
    )(y2, gath2, aff2, affm1, aW2.T, jnp.broadcast_to(ab2, (8, 32)),
      mW2.T, jnp.broadcast_to(mb2, (8, 32)))

    sm2, qm2 = stats_m2[0, :32], stats_m2[0, 32:64]
    mm2 = sm2 / N
    vm2 = qm2 / N - mm2 * mm2
    scm2 = mg2 / jnp.sqrt(vm2 + 1e-5)
    affm2 = jnp.stack([scm2, mbe2 - mm2 * scm2])

    out = pl.pallas_call(
        _tc4_body,
        grid=(G,),
        in_specs=[
            pl.BlockSpec((TN, 32), lambda i: (i, 0)),
            _full((2, 32)),
        ],
        out_specs=pl.BlockSpec((TN, 32), lambda i: (i, 0)),
        out_shape=jax.ShapeDtypeStruct((N, 32), f32),
    )(y_m2, affm2)

    return out.T.reshape(1, 32, N, 1)


# speedup vs baseline: 10.2394x; 1.0006x over previous
import functools

import jax
import jax.numpy as jnp
from jax import lax
from jax.experimental import pallas as pl
from jax.experimental.pallas import tpu as pltpu
from jax.experimental.pallas import tpu_sc as plsc

N = 50000
K = 16
E = N * K
TN = 400
RT = TN * K
G = N // TN

_NC = 2
_NS = 16
_NW = _NC * _NS
_CH = 1000

_HI = jax.lax.Precision.HIGHEST
_LO = jax.lax.Precision.DEFAULT


def _sc_gather(table, idx):
    e = idx.shape[0]
    d = table.shape[1]
    bpw = e // _NW
    nch = bpw // _CH
    mesh = plsc.VectorSubcoreMesh(core_axis_name="c", subcore_axis_name="s")

    @functools.partial(
        pl.kernel,
        mesh=mesh,
        out_type=jax.ShapeDtypeStruct((e, d), jnp.float32),
        compiler_params=pltpu.CompilerParams(use_tc_tiling_on_sc=False),
        scratch_types=[
            pltpu.VMEM((_CH,), jnp.int32),
            pltpu.VMEM((_CH, d), jnp.float32),
            pltpu.SemaphoreType.DMA,
        ],
    )
    def k(table_hbm, idx_hbm, out_hbm, idx_v, rows_v, sem):
        wid = lax.axis_index("s") * _NC + lax.axis_index("c")
        base = wid * bpw

        def body(c, carry):
            off = base + c * _CH
            pltpu.sync_copy(idx_hbm.at[pl.ds(off, _CH)], idx_v)
            pltpu.async_copy(table_hbm.at[idx_v], rows_v, sem).wait()
            pltpu.sync_copy(rows_v, out_hbm.at[pl.ds(off, _CH)])
            return carry

        lax.fori_loop(0, nch, body, 0)

    return k(table, idx)


def _stats_accum(ref, vals):
    s = jnp.sum(vals, axis=0, keepdims=True)
    q = jnp.sum(vals * vals, axis=0, keepdims=True)
    part = jnp.broadcast_to(jnp.concatenate([s, q], axis=1), ref.shape)
    i = pl.program_id(0)

    @pl.when(i == 0)
    def _():
        ref[...] = part

    @pl.when(i > 0)
    def _():
        ref[...] = ref[...] + part


def _tc1_body(gath_ref, xyz_ref, w1t_ref, b1_ref, y1_ref, stats_ref):
    xj = gath_ref[...][:, 0:3]
    xi = jnp.broadcast_to(xyz_ref[...][:, None, :], (TN, K, 3)).reshape(RT, 3)
    rel = xi - xj
    dis = jnp.sqrt(jnp.sum(rel * rel, axis=1, keepdims=True))
    rp = jnp.concatenate([dis, rel, xi, xj], axis=1)
    y1 = jnp.dot(rp, w1t_ref[...], precision=_LO) + b1_ref[0:1, :]
    y1_ref[...] = y1
    _stats_accum(stats_ref, y1)


def _softmax_pool(f_cat, att):
    a3 = att.reshape(TN, K, att.shape[1])
    f3 = f_cat.reshape(TN, K, f_cat.shape[1])
    m = jnp.max(a3, axis=1, keepdims=True)
    ex = jnp.exp(a3 - m)
    sm = jnp.sum(ex, axis=1, keepdims=True)
    return jnp.sum(f3 * (ex / sm), axis=1)


def _tc2_body(y1_ref, gath_ref, aff1_ref, awc_ref, abc_ref, mw1t_ref, mb1_ref,
              ym1_ref, y2_ref, stats_m1_ref, stats_y2_ref):
    f_xyz = jnp.maximum(
        y1_ref[...] * aff1_ref[0:1, :] + aff1_ref[1:2, :], 0.0)
    f_cat = jnp.concatenate([gath_ref[...][:, 16:32], f_xyz], axis=1)
    av = jnp.dot(f_cat, awc_ref[...], precision=_LO) + abc_ref[0:1, :]
    att = av[:, 0:32]
    y2 = av[:, 32:48]
    agg = _softmax_pool(f_cat, att)
    y_m1 = jnp.dot(agg, mw1t_ref[...], precision=_HI) + mb1_ref[0:1, :]
    ym1_ref[...] = y_m1
    y2_ref[...] = y2
    _stats_accum(stats_m1_ref, y_m1)
    _stats_accum(stats_y2_ref, y2)


def _tc3_body(y2_ref, gath2_ref, aff2_ref, affm1_ref, aw2t_ref, ab2_ref,
              mw2t_ref, mb2_ref, ym2_ref, stats_m2_ref):
    f_xyz2 = jnp.maximum(y2_ref[...] * aff2_ref[0:1, :] + aff2_ref[1:2, :], 0.0)
    f_nb2 = jnp.maximum(gath2_ref[...] * affm1_ref[0:1, :] + affm1_ref[1:2, :], 0.0)
    f_cat2 = jnp.concatenate([f_nb2, f_xyz2], axis=1)
    att2 = jnp.dot(f_cat2, aw2t_ref[...], precision=_LO) + ab2_ref[0:1, :]
    agg2 = _softmax_pool(f_cat2, att2)
    y_m2 = jnp.dot(agg2, mw2t_ref[...], precision=_HI) + mb2_ref[0:1, :]
    ym2_ref[...] = y_m2
    _stats_accum(stats_m2_ref, y_m2)


def _tc4_body(ym2_ref, affm2_ref, out_ref):
    out_ref[...] = jnp.maximum(
        ym2_ref[...] * affm2_ref[0:1, :] + affm2_ref[1:2, :], 0.0)


def _full(shape):
    return pl.BlockSpec(shape, lambda i: (0, 0))


def kernel(xyz, feature, neigh_idx, W1, b1, g1, be1, aW1, ab1, mW1, mb1,
           mg1, mbe1, W2, b2, g2, be2, aW2, ab2, mW2, mb2, mg2, mbe2):
    f32 = jnp.float32
    P = xyz[0].astype(f32)
    feat = feature[0, :, :, 0].T.astype(f32)
    table1 = jnp.concatenate(
        [P, jnp.zeros((N, 13), f32), feat], axis=1)
    idx = neigh_idx.reshape(E).astype(jnp.int32)

    gath1 = _sc_gather(table1, idx)

    y1, stats1 = pl.pallas_call(
        _tc1_body,
        grid=(G,),
        in_specs=[
            pl.BlockSpec((RT, 32), lambda i: (i, 0)),
            pl.BlockSpec((TN, 3), lambda i: (i, 0)),
            _full((10, 16)),
            _full((8, 16)),
        ],
        out_specs=[
            pl.BlockSpec((RT, 16), lambda i: (i, 0)),
            _full((8, 32)),
        ],
        out_shape=[
            jax.ShapeDtypeStruct((E, 16), f32),
            jax.ShapeDtypeStruct((8, 32), f32),
        ],
    )(gath1, P, W1.T, jnp.broadcast_to(b1, (8, 16)))

    s1, q1 = stats1[0, :16], stats1[0, 16:32]
    m1 = s1 / E
    v1 = q1 / E - m1 * m1
    sc1 = g1 / jnp.sqrt(v1 + 1e-5)
    aff1 = jnp.stack([sc1, be1 - m1 * sc1])

    aWc = jnp.concatenate(
        [aW1.T, jnp.concatenate([jnp.zeros((16, 16), f32), W2.T], axis=0)],
        axis=1)
    abc = jnp.concatenate([ab1, b2])

    y_m1, y2, stats_m1, stats_y2 = pl.pallas_call(
        _tc2_body,
        grid=(G,),
        in_specs=[
            pl.BlockSpec((RT, 16), lambda i: (i, 0)),
            pl.BlockSpec((RT, 32), lambda i: (i, 0)),
            _full((2, 16)),
            _full((32, 48)),
            _full((8, 48)),
            _full((32, 16)),
            _full((8, 16)),
        ],
        out_specs=[
            pl.BlockSpec((TN, 16), lambda i: (i, 0)),
            pl.BlockSpec((RT, 16), lambda i: (i, 0)),
            _full((8, 32)),
            _full((8, 32)),
        ],
        out_shape=[
            jax.ShapeDtypeStruct((N, 16), f32),
            jax.ShapeDtypeStruct((E, 16), f32),
            jax.ShapeDtypeStruct((8, 32), f32),
            jax.ShapeDtypeStruct((8, 32), f32),
        ],
    )(y1, gath1, aff1, aWc, jnp.broadcast_to(abc, (8, 48)),
      mW1.T, jnp.broadcast_to(mb1, (8, 16)))

    sm, qm = stats_m1[0, :16], stats_m1[0, 16:32]
    mm = sm / N
    vm = qm / N - mm * mm
    scm = mg1 / jnp.sqrt(vm + 1e-5)
    affm1 = jnp.stack([scm, mbe1 - mm * scm])

    s2, q2 = stats_y2[0, :16], stats_y2[0, 16:32]
    m2 = s2 / E
    v2 = q2 / E - m2 * m2
    sc2 = g2 / jnp.sqrt(v2 + 1e-5)
    aff2 = jnp.stack([sc2, be2 - m2 * sc2])

    gath2 = _sc_gather(y_m1, idx)

    y_m2, stats_m2 = pl.pallas_call(
        _tc3_body,
        grid=(G,),
        in_specs=[
            pl.BlockSpec((RT, 16), lambda i: (i, 0)),
            pl.BlockSpec((RT, 16), lambda i: (i, 0)),
            _full((2, 16)),
            _full((2, 16)),
            _full((32, 32)),
            _full((8, 32)),
            _full((32, 32)),
            _full((8, 32)),
        ],
        out_specs=[
            pl.BlockSpec((TN, 32), lambda i: (i, 0)),
            _full((8, 64)),
        ],
        out_shape=[
            jax.ShapeDtypeStruct((N, 32), f32),
            jax.ShapeDtypeStruct((8, 64), f32),
        ],
    )(y2, gath2, aff2, affm1, aW2.T, jnp.broadcast_to(ab2, (8, 32)),
      mW2.T, jnp.broadcast_to(mb2, (8, 32)))

    sm2, qm2 = stats_m2[0, :32], stats_m2[0, 32:64]
    mm2 = sm2 / N
    vm2 = qm2 / N - mm2 * mm2
    scm2 = mg2 / jnp.sqrt(vm2 + 1e-5)
    affm2 = jnp.stack([scm2, mbe2 - mm2 * scm2])

    out = pl.pallas_call(
        _tc4_body,
        grid=(G,),
        in_specs=[
            pl.BlockSpec((TN, 32), lambda i: (i, 0)),
            _full((2, 32)),
        ],
        out_specs=pl.BlockSpec((TN, 32), lambda i: (i, 0)),
        out_shape=jax.ShapeDtypeStruct((N, 32), f32),
    )(y_m2, affm2)

    return out.T.reshape(1, 32, N, 1)
